# Initial kernel scaffold; baseline (speedup 1.0000x reference)
import jax
import jax.numpy as jnp
from jax.experimental import pallas as pl


def kernel(x, edge_index, batch, W1, b1, W2, b2, Wl1, bl1, Wl2, bl2):
    raise NotImplementedError("write your pallas kernel here")


# speedup vs baseline: 12.0570x; 12.0570x over previous
import functools

import jax
import jax.numpy as jnp
from jax import lax
from jax.experimental import pallas as pl
from jax.experimental.pallas import tpu as pltpu
from jax.experimental.pallas import tpu_sc as plsc

N = 10000
E = 320000
IN_CH = 128
HID = 64
G = 64

NC, NS = 2, 16
NW = NC * NS
CH = 128
SUP = 8
NPAD = 10240
EROWS = 2560
ROWS_PER_W = EROWS // NW
NSUP = ROWS_PER_W // SUP
STRIPE = NPAD // NS

def _worker_id():
    return lax.axis_index("s") * NC + lax.axis_index("c")


def _mesh():
    return plsc.VectorSubcoreMesh(
        core_axis_name="c", subcore_axis_name="s",
        num_cores=NC, num_subcores=NS)


@functools.cache
def _make_deg_kernel():
    return pl.kernel(
        _deg_body,
        out_type=jax.ShapeDtypeStruct((NC, NPAD), jnp.float32),
        mesh=_mesh(),
        scratch_types=[
            pltpu.VMEM_SHARED((NPAD,), jnp.float32),
            pltpu.VMEM((SUP, CH), jnp.int32),
            pltpu.VMEM((CH,), jnp.float32),
            pltpu.VMEM((STRIPE,), jnp.float32),
        ],
    )


def _deg_body(dst_hbm, out_hbm, deg_sh, dst_v, ones_v, zb_v):
    c = lax.axis_index("c")
    s = lax.axis_index("s")
    w = _worker_id()

    def fill(i, _):
        zb_v[pl.ds(i * 16, 16)] = jnp.zeros((16,), jnp.float32)
        return 0
    lax.fori_loop(0, STRIPE // 16, fill, 0)
    for j in range(CH // 16):
        ones_v[pl.ds(j * 16, 16)] = jnp.ones((16,), jnp.float32)
    pltpu.sync_copy(zb_v, deg_sh.at[pl.ds(s * STRIPE, STRIPE)])
    plsc.subcore_barrier()

    def step(i, _):
        base = w * ROWS_PER_W + i * SUP
        pltpu.sync_copy(dst_hbm.at[pl.ds(base, SUP)], dst_v)
        for j in range(SUP):
            pltpu.sync_copy(ones_v, deg_sh.at[dst_v.at[j]], add=True)
        return 0
    lax.fori_loop(0, NSUP, step, 0)

    plsc.subcore_barrier()
    @pl.when(s == 0)
    def _():
        pltpu.sync_copy(deg_sh, out_hbm.at[c])


@functools.cache
def _make_agg_kernel():
    return pl.kernel(
        _agg_body,
        out_type=jax.ShapeDtypeStruct((NC, NPAD, HID), jnp.float32),
        mesh=_mesh(),
        compiler_params=pltpu.CompilerParams(use_tc_tiling_on_sc=False),
        scratch_types=[
            pltpu.VMEM_SHARED((NPAD, HID), jnp.float32),
            pltpu.VMEM((SUP, CH), jnp.int32),
            pltpu.VMEM((SUP, CH), jnp.int32),
            pltpu.VMEM((CH, HID), jnp.float32),
        ],
    )


def _agg_body(hp_hbm, src_hbm, dst_hbm, out_hbm, acc_sh, src_v, dst_v, rows_v):
    c = lax.axis_index("c")
    s = lax.axis_index("s")
    w = _worker_id()

    pltpu.sync_copy(hp_hbm.at[pl.ds(s * STRIPE, STRIPE)],
                    acc_sh.at[pl.ds(s * STRIPE, STRIPE)])
    plsc.subcore_barrier()

    def step(i, _):
        base = w * ROWS_PER_W + i * SUP
        pltpu.sync_copy(src_hbm.at[pl.ds(base, SUP)], src_v)
        pltpu.sync_copy(dst_hbm.at[pl.ds(base, SUP)], dst_v)
        for j in range(SUP):
            pltpu.sync_copy(hp_hbm.at[src_v.at[j]], rows_v)
            pltpu.sync_copy(rows_v, acc_sh.at[dst_v.at[j]], add=True)
        return 0
    lax.fori_loop(0, NSUP, step, 0)

    plsc.subcore_barrier()
    pltpu.sync_copy(acc_sh.at[pl.ds(s * STRIPE, STRIPE)],
                    out_hbm.at[c, pl.ds(s * STRIPE, STRIPE)])


def _prep_body(x_ref, w1_ref, degt_ref, hp_ref):
    dsum = degt_ref[:, 0:1] + degt_ref[:, 1:2] + 1.0
    dis = lax.rsqrt(dsum)
    hp_ref[...] = jnp.dot(x_ref[...], w1_ref[...],
                          preferred_element_type=jnp.float32) * dis


def _mid_body(a_ref, hp_ref, degt_ref, b1_ref, w2_ref, hp2_ref):
    dsum = degt_ref[:, 0:1] + degt_ref[:, 1:2] + 1.0
    dis = lax.rsqrt(dsum)
    tot = a_ref[0] + a_ref[1] - hp_ref[...]
    h1 = jnp.maximum(tot * dis + b1_ref[...], 0.0)
    hp2_ref[...] = jnp.dot(h1, w2_ref[...],
                           preferred_element_type=jnp.float32) * dis


def _head_body(a_ref, hp_ref, degt_ref, b2_ref, batch_ref,
               wl1_ref, bl1_ref, wl2_ref, bl2_ref, out_ref):
    dsum = degt_ref[:, 0:1] + degt_ref[:, 1:2] + 1.0
    dis = lax.rsqrt(dsum)
    tot = a_ref[0] + a_ref[1] - hp_ref[...]
    h2 = jnp.maximum(tot * dis + b2_ref[...], 0.0)
    gid = lax.broadcasted_iota(jnp.int32, (N, G), 1)
    oh = (batch_ref[...] == gid).astype(jnp.float32)
    sums = lax.dot_general(oh, h2, (((0,), (0,)), ((), ())),
                           preferred_element_type=jnp.float32)
    counts = lax.dot_general(oh, jnp.ones((N, 1), jnp.float32),
                             (((0,), (0,)), ((), ())),
                             preferred_element_type=jnp.float32)
    pooled = sums / jnp.maximum(counts, 1.0)
    h = jnp.maximum(jnp.dot(pooled, wl1_ref[...],
                            preferred_element_type=jnp.float32) + bl1_ref[...], 0.0)
    logits = jnp.dot(h, wl2_ref[...],
                     preferred_element_type=jnp.float32) + bl2_ref[...]
    out_ref[...] = 1.0 / (1.0 + jnp.exp(-logits))


def kernel(x, edge_index, batch, W1, b1, W2, b2, Wl1, bl1, Wl2, bl2):
    pad = EROWS * CH - E
    fill = jnp.full((pad,), N, dtype=jnp.int32)
    src2d = jnp.concatenate([edge_index[0], fill]).reshape(EROWS, CH)
    dst2d = jnp.concatenate([edge_index[1], fill]).reshape(EROWS, CH)
    batch2d = batch.reshape(N, 1)

    degp = _make_deg_kernel()(dst2d)
    degt = degp[:, :N].T

    hp1 = pl.pallas_call(
        _prep_body,
        out_shape=jax.ShapeDtypeStruct((NPAD, HID), jnp.float32),
    )(jnp.pad(x, ((0, NPAD - N), (0, 0))), W1,
      jnp.pad(degt, ((0, NPAD - N), (0, 0))))

    a1 = _make_agg_kernel()(hp1, src2d, dst2d)

    hp2 = pl.pallas_call(
        _mid_body,
        out_shape=jax.ShapeDtypeStruct((NPAD, HID), jnp.float32),
    )(a1, hp1, jnp.pad(degt, ((0, NPAD - N), (0, 0))),
      b1.reshape(1, HID), W2)

    a2 = _make_agg_kernel()(hp2, src2d, dst2d)

    out = pl.pallas_call(
        _head_body,
        out_shape=jax.ShapeDtypeStruct((G, 1), jnp.float32),
    )(a2[:, :N, :], hp2[:N], degt, b2.reshape(1, HID), batch2d,
      Wl1, bl1.reshape(1, HID), Wl2, bl2.reshape(1, 1))
    return out.reshape(-1)


# speedup vs baseline: 14.1484x; 1.1735x over previous
import functools

import jax
import jax.numpy as jnp
from jax import lax
from jax.experimental import pallas as pl
from jax.experimental.pallas import tpu as pltpu
from jax.experimental.pallas import tpu_sc as plsc

N = 10000
E = 320000
IN_CH = 128
HID = 64
G = 64

NC, NS = 2, 16
NW = NC * NS
CH = 128
SUP = 8
NPAD = 10240
EROWS = 2560
ROWS_PER_W = EROWS // NW
NSUP = ROWS_PER_W // SUP
STRIPE = NPAD // NS

def _worker_id():
    return lax.axis_index("s") * NC + lax.axis_index("c")


def _mesh():
    return plsc.VectorSubcoreMesh(
        core_axis_name="c", subcore_axis_name="s",
        num_cores=NC, num_subcores=NS)


@functools.cache
def _make_deg_kernel():
    return pl.kernel(
        _deg_body,
        out_type=jax.ShapeDtypeStruct((NC, NPAD), jnp.float32),
        mesh=_mesh(),
        scratch_types=[
            pltpu.VMEM_SHARED((NPAD,), jnp.float32),
            pltpu.VMEM((SUP, CH), jnp.int32),
            pltpu.VMEM((CH,), jnp.float32),
            pltpu.VMEM((STRIPE,), jnp.float32),
        ],
    )


def _deg_body(dst_hbm, out_hbm, deg_sh, dst_v, ones_v, zb_v):
    c = lax.axis_index("c")
    s = lax.axis_index("s")
    w = _worker_id()

    def fill(i, _):
        zb_v[pl.ds(i * 16, 16)] = jnp.zeros((16,), jnp.float32)
        return 0
    lax.fori_loop(0, STRIPE // 16, fill, 0)
    for j in range(CH // 16):
        ones_v[pl.ds(j * 16, 16)] = jnp.ones((16,), jnp.float32)
    pltpu.sync_copy(zb_v, deg_sh.at[pl.ds(s * STRIPE, STRIPE)])
    plsc.subcore_barrier()

    def step(i, _):
        base = w * ROWS_PER_W + i * SUP
        pltpu.sync_copy(dst_hbm.at[pl.ds(base, SUP)], dst_v)
        for j in range(SUP):
            pltpu.sync_copy(ones_v, deg_sh.at[dst_v.at[j]], add=True)
        return 0
    lax.fori_loop(0, NSUP, step, 0)

    plsc.subcore_barrier()
    @pl.when(s == 0)
    def _():
        pltpu.sync_copy(deg_sh, out_hbm.at[c])


GRP = 4
NGRP = ROWS_PER_W // GRP


@functools.cache
def _make_agg_kernel():
    return pl.kernel(
        _agg_body,
        out_type=jax.ShapeDtypeStruct((NC, NPAD, HID), jnp.float32),
        mesh=_mesh(),
        compiler_params=pltpu.CompilerParams(use_tc_tiling_on_sc=False),
        scratch_types=[
            pltpu.VMEM_SHARED((NPAD, HID), jnp.float32),
            pltpu.VMEM((ROWS_PER_W, CH), jnp.int32),
            pltpu.VMEM((ROWS_PER_W, CH), jnp.int32),
            pltpu.VMEM((2 * GRP, CH, HID), jnp.float32),
            pltpu.SemaphoreType.DMA,
            pltpu.SemaphoreType.DMA,
        ],
    )


def _agg_body(hp_hbm, src_hbm, dst_hbm, out_hbm,
              acc_sh, src_v, dst_v, rows_v, sem_g, sem_s):
    c = lax.axis_index("c")
    s = lax.axis_index("s")
    w = _worker_id()
    base = w * ROWS_PER_W

    def drain(sem, n):
        for _ in range(n):
            pltpu.make_async_copy(
                hp_hbm.at[pl.ds(0, CH)], rows_v.at[0], sem).wait()

    pltpu.sync_copy(src_hbm.at[pl.ds(base, ROWS_PER_W)], src_v)
    pltpu.sync_copy(dst_hbm.at[pl.ds(base, ROWS_PER_W)], dst_v)
    for j in range(GRP):
        pltpu.async_copy(hp_hbm.at[src_v.at[j]], rows_v.at[j], sem_g)
    pltpu.sync_copy(hp_hbm.at[pl.ds(s * STRIPE, STRIPE)],
                    acc_sh.at[pl.ds(s * STRIPE, STRIPE)])
    plsc.subcore_barrier()

    def grp_step(g, _):
        nxt = (g + 1) % 2
        @pl.when(g >= 1)
        def _():
            drain(sem_s, GRP)

        @pl.when(g <= NGRP - 2)
        def _():
            for j in range(GRP):
                pltpu.async_copy(hp_hbm.at[src_v.at[(g + 1) * GRP + j]],
                                 rows_v.at[nxt * GRP + j], sem_g)
        drain(sem_g, GRP)
        cur = g % 2
        for j in range(GRP):
            pltpu.async_copy(rows_v.at[cur * GRP + j],
                             acc_sh.at[dst_v.at[g * GRP + j]], sem_s,
                             add=True)
        return 0
    lax.fori_loop(0, NGRP, grp_step, 0)
    drain(sem_s, GRP)

    plsc.subcore_barrier()
    pltpu.sync_copy(acc_sh.at[pl.ds(s * STRIPE, STRIPE)],
                    out_hbm.at[c, pl.ds(s * STRIPE, STRIPE)])


def _prep_body(x_ref, w1_ref, degt_ref, hp_ref):
    dsum = degt_ref[:, 0:1] + degt_ref[:, 1:2] + 1.0
    dis = lax.rsqrt(dsum)
    hp_ref[...] = jnp.dot(x_ref[...], w1_ref[...],
                          preferred_element_type=jnp.float32) * dis


def _mid_body(a_ref, hp_ref, degt_ref, b1_ref, w2_ref, hp2_ref):
    dsum = degt_ref[:, 0:1] + degt_ref[:, 1:2] + 1.0
    dis = lax.rsqrt(dsum)
    tot = a_ref[0] + a_ref[1] - hp_ref[...]
    h1 = jnp.maximum(tot * dis + b1_ref[...], 0.0)
    hp2_ref[...] = jnp.dot(h1, w2_ref[...],
                           preferred_element_type=jnp.float32) * dis


def _head_body(a_ref, hp_ref, degt_ref, b2_ref, batch_ref,
               wl1_ref, bl1_ref, wl2_ref, bl2_ref, out_ref):
    dsum = degt_ref[:, 0:1] + degt_ref[:, 1:2] + 1.0
    dis = lax.rsqrt(dsum)
    tot = a_ref[0] + a_ref[1] - hp_ref[...]
    h2 = jnp.maximum(tot * dis + b2_ref[...], 0.0)
    gid = lax.broadcasted_iota(jnp.int32, (N, G), 1)
    oh = (batch_ref[...] == gid).astype(jnp.float32)
    sums = lax.dot_general(oh, h2, (((0,), (0,)), ((), ())),
                           preferred_element_type=jnp.float32)
    counts = lax.dot_general(oh, jnp.ones((N, 1), jnp.float32),
                             (((0,), (0,)), ((), ())),
                             preferred_element_type=jnp.float32)
    pooled = sums / jnp.maximum(counts, 1.0)
    h = jnp.maximum(jnp.dot(pooled, wl1_ref[...],
                            preferred_element_type=jnp.float32) + bl1_ref[...], 0.0)
    logits = jnp.dot(h, wl2_ref[...],
                     preferred_element_type=jnp.float32) + bl2_ref[...]
    out_ref[...] = 1.0 / (1.0 + jnp.exp(-logits))


def kernel(x, edge_index, batch, W1, b1, W2, b2, Wl1, bl1, Wl2, bl2):
    pad = EROWS * CH - E
    fill = jnp.full((pad,), N, dtype=jnp.int32)
    src2d = jnp.concatenate([edge_index[0], fill]).reshape(EROWS, CH)
    dst2d = jnp.concatenate([edge_index[1], fill]).reshape(EROWS, CH)
    batch2d = batch.reshape(N, 1)

    degp = _make_deg_kernel()(dst2d)
    degt = degp[:, :N].T

    hp1 = pl.pallas_call(
        _prep_body,
        out_shape=jax.ShapeDtypeStruct((NPAD, HID), jnp.float32),
    )(jnp.pad(x, ((0, NPAD - N), (0, 0))), W1,
      jnp.pad(degt, ((0, NPAD - N), (0, 0))))

    a1 = _make_agg_kernel()(hp1, src2d, dst2d)

    hp2 = pl.pallas_call(
        _mid_body,
        out_shape=jax.ShapeDtypeStruct((NPAD, HID), jnp.float32),
    )(a1, hp1, jnp.pad(degt, ((0, NPAD - N), (0, 0))),
      b1.reshape(1, HID), W2)

    a2 = _make_agg_kernel()(hp2, src2d, dst2d)

    out = pl.pallas_call(
        _head_body,
        out_shape=jax.ShapeDtypeStruct((G, 1), jnp.float32),
    )(a2[:, :N, :], hp2[:N], degt, b2.reshape(1, HID), batch2d,
      Wl1, bl1.reshape(1, HID), Wl2, bl2.reshape(1, 1))
    return out.reshape(-1)


# speedup vs baseline: 34.9722x; 2.4718x over previous
import functools

import jax
import jax.numpy as jnp
from jax import lax
from jax.experimental import pallas as pl
from jax.experimental.pallas import tpu as pltpu
from jax.experimental.pallas import tpu_sc as plsc

N = 10000
E = 320000
IN_CH = 128
HID = 64
G = 64

NC, NS = 2, 16
NW = NC * NS
CH = 128
SUP = 8
NPAD = 10240
EROWS = 2560
ROWS_PER_W = EROWS // NW
NSUP = ROWS_PER_W // SUP
STRIPE = NPAD // NS

def _worker_id():
    return lax.axis_index("s") * NC + lax.axis_index("c")


def _mesh():
    return plsc.VectorSubcoreMesh(
        core_axis_name="c", subcore_axis_name="s",
        num_cores=NC, num_subcores=NS)


@functools.cache
def _make_deg_kernel():
    return pl.kernel(
        _deg_body,
        out_type=jax.ShapeDtypeStruct((NC, NPAD), jnp.float32),
        mesh=_mesh(),
        scratch_types=[
            pltpu.VMEM_SHARED((NPAD,), jnp.float32),
            pltpu.VMEM((SUP, CH), jnp.int32),
            pltpu.VMEM((CH,), jnp.float32),
            pltpu.VMEM((STRIPE,), jnp.float32),
        ],
    )


def _deg_body(dst_hbm, out_hbm, deg_sh, dst_v, ones_v, zb_v):
    c = lax.axis_index("c")
    s = lax.axis_index("s")
    w = _worker_id()

    def fill(i, _):
        zb_v[pl.ds(i * 16, 16)] = jnp.zeros((16,), jnp.float32)
        return 0
    lax.fori_loop(0, STRIPE // 16, fill, 0)
    for j in range(CH // 16):
        ones_v[pl.ds(j * 16, 16)] = jnp.ones((16,), jnp.float32)
    pltpu.sync_copy(zb_v, deg_sh.at[pl.ds(s * STRIPE, STRIPE)])
    plsc.subcore_barrier()

    def step(i, _):
        base = w * ROWS_PER_W + i * SUP
        pltpu.sync_copy(dst_hbm.at[pl.ds(base, SUP)], dst_v)
        for j in range(SUP):
            pltpu.sync_copy(ones_v, deg_sh.at[dst_v.at[j]], add=True)
        return 0
    lax.fori_loop(0, NSUP, step, 0)

    plsc.subcore_barrier()
    @pl.when(s == 0)
    def _():
        pltpu.sync_copy(deg_sh, out_hbm.at[c])


GRP = 2
NGRP = ROWS_PER_W // GRP


@functools.cache
def _make_agg_kernel():
    return pl.kernel(
        _agg_body,
        out_type=jax.ShapeDtypeStruct((NC, NPAD, HID), jnp.float32),
        mesh=_mesh(),
        compiler_params=pltpu.CompilerParams(use_tc_tiling_on_sc=False),
        scratch_types=[
            pltpu.VMEM_SHARED((NPAD, HID), jnp.float32),
            pltpu.VMEM_SHARED((NPAD, HID), jnp.float32),
            pltpu.VMEM((3, GRP, CH), jnp.int32),
            pltpu.VMEM((3, GRP, CH), jnp.int32),
            pltpu.VMEM((2 * GRP, CH, HID), jnp.float32),
            pltpu.SemaphoreType.DMA,
            pltpu.SemaphoreType.DMA,
            pltpu.SemaphoreType.DMA,
        ],
    )


def _agg_body(hp_hbm, src_hbm, dst_hbm, out_hbm,
              acc_sh, hp_sh, src_v, dst_v, rows_v, sem_i, sem_g, sem_s):
    c = lax.axis_index("c")
    s = lax.axis_index("s")
    w = _worker_id()
    base = w * ROWS_PER_W

    def drain(sem, n, idx_shape=False):
        for _ in range(n):
            if idx_shape:
                pltpu.make_async_copy(
                    src_hbm.at[pl.ds(0, GRP)], src_v.at[0], sem).wait()
            else:
                pltpu.make_async_copy(
                    hp_hbm.at[pl.ds(0, CH)], rows_v.at[0], sem).wait()

    def load_idx(g, slot):
        pltpu.async_copy(src_hbm.at[pl.ds(base + g * GRP, GRP)],
                         src_v.at[slot], sem_i)
        pltpu.async_copy(dst_hbm.at[pl.ds(base + g * GRP, GRP)],
                         dst_v.at[slot], sem_i)

    def fire_gathers(g, rowset):
        slot = lax.rem(g, 3)
        for j in range(GRP):
            pltpu.async_copy(hp_sh.at[src_v.at[slot, j]],
                             rows_v.at[rowset * GRP + j], sem_g)

    load_idx(0, 0)

    def zfill(i, _):
        rows_v[0, pl.ds(i * 16, 16), 0:HID] = jnp.zeros((16, HID), jnp.float32)
        return 0
    lax.fori_loop(0, CH // 16, zfill, 0)
    for k in range(STRIPE // CH):
        pltpu.sync_copy(rows_v.at[0], acc_sh.at[pl.ds(s * STRIPE + k * CH, CH)])
    pltpu.sync_copy(hp_hbm.at[pl.ds(s * STRIPE, STRIPE)],
                    hp_sh.at[pl.ds(s * STRIPE, STRIPE)])
    plsc.subcore_barrier()

    drain(sem_i, 2, idx_shape=True)
    fire_gathers(0, 0)
    load_idx(1, 1)

    def grp_step(g, _):
        nxt = (g + 1) % 2
        @pl.when(g >= 1)
        def _():
            drain(sem_s, GRP)

        @pl.when(g <= NGRP - 2)
        def _():
            drain(sem_i, 2, idx_shape=True)
            fire_gathers(g + 1, nxt)

        @pl.when(g <= NGRP - 3)
        def _():
            load_idx(g + 2, lax.rem(g + 2, 3))

        drain(sem_g, GRP)
        cur = g % 2
        slot = lax.rem(g, 3)
        for j in range(GRP):
            pltpu.async_copy(rows_v.at[cur * GRP + j],
                             acc_sh.at[dst_v.at[slot, j]], sem_s,
                             add=True)
        return 0
    lax.fori_loop(0, NGRP, grp_step, 0)
    drain(sem_s, GRP)

    plsc.subcore_barrier()
    pltpu.sync_copy(acc_sh.at[pl.ds(s * STRIPE, STRIPE)],
                    out_hbm.at[c, pl.ds(s * STRIPE, STRIPE)])


def _prep_body(x_ref, w1_ref, degt_ref, hp_ref):
    dsum = degt_ref[:, 0:1] + degt_ref[:, 1:2] + 1.0
    dis = lax.rsqrt(dsum)
    hp_ref[...] = jnp.dot(x_ref[...], w1_ref[...],
                          preferred_element_type=jnp.float32) * dis


def _mid_body(a_ref, hp_ref, degt_ref, b1_ref, w2_ref, hp2_ref):
    dsum = degt_ref[:, 0:1] + degt_ref[:, 1:2] + 1.0
    dis = lax.rsqrt(dsum)
    tot = a_ref[0] + a_ref[1] + hp_ref[...]
    h1 = jnp.maximum(tot * dis + b1_ref[...], 0.0)
    hp2_ref[...] = jnp.dot(h1, w2_ref[...],
                           preferred_element_type=jnp.float32) * dis


def _head_body(a_ref, hp_ref, degt_ref, b2_ref, batch_ref,
               wl1_ref, bl1_ref, wl2_ref, bl2_ref, out_ref):
    dsum = degt_ref[:, 0:1] + degt_ref[:, 1:2] + 1.0
    dis = lax.rsqrt(dsum)
    tot = a_ref[0] + a_ref[1] + hp_ref[...]
    h2 = jnp.maximum(tot * dis + b2_ref[...], 0.0)
    gid = lax.broadcasted_iota(jnp.int32, (N, G), 1)
    oh = (batch_ref[...] == gid).astype(jnp.float32)
    sums = lax.dot_general(oh, h2, (((0,), (0,)), ((), ())),
                           preferred_element_type=jnp.float32)
    counts = lax.dot_general(oh, jnp.ones((N, 1), jnp.float32),
                             (((0,), (0,)), ((), ())),
                             preferred_element_type=jnp.float32)
    pooled = sums / jnp.maximum(counts, 1.0)
    h = jnp.maximum(jnp.dot(pooled, wl1_ref[...],
                            preferred_element_type=jnp.float32) + bl1_ref[...], 0.0)
    logits = jnp.dot(h, wl2_ref[...],
                     preferred_element_type=jnp.float32) + bl2_ref[...]
    out_ref[...] = 1.0 / (1.0 + jnp.exp(-logits))


def kernel(x, edge_index, batch, W1, b1, W2, b2, Wl1, bl1, Wl2, bl2):
    pad = EROWS * CH - E
    fill = jnp.full((pad,), N, dtype=jnp.int32)
    src2d = jnp.concatenate([edge_index[0], fill]).reshape(EROWS, CH)
    dst2d = jnp.concatenate([edge_index[1], fill]).reshape(EROWS, CH)
    batch2d = batch.reshape(N, 1)

    degp = _make_deg_kernel()(dst2d)
    degt = degp[:, :N].T

    hp1 = pl.pallas_call(
        _prep_body,
        out_shape=jax.ShapeDtypeStruct((NPAD, HID), jnp.float32),
    )(jnp.pad(x, ((0, NPAD - N), (0, 0))), W1,
      jnp.pad(degt, ((0, NPAD - N), (0, 0))))

    a1 = _make_agg_kernel()(hp1, src2d, dst2d)

    hp2 = pl.pallas_call(
        _mid_body,
        out_shape=jax.ShapeDtypeStruct((NPAD, HID), jnp.float32),
    )(a1, hp1, jnp.pad(degt, ((0, NPAD - N), (0, 0))),
      b1.reshape(1, HID), W2)

    a2 = _make_agg_kernel()(hp2, src2d, dst2d)

    out = pl.pallas_call(
        _head_body,
        out_shape=jax.ShapeDtypeStruct((G, 1), jnp.float32),
    )(a2[:, :N, :], hp2[:N], degt, b2.reshape(1, HID), batch2d,
      Wl1, bl1.reshape(1, HID), Wl2, bl2.reshape(1, 1))
    return out.reshape(-1)


# speedup vs baseline: 37.7311x; 1.0789x over previous
import functools

import jax
import jax.numpy as jnp
from jax import lax
from jax.experimental import pallas as pl
from jax.experimental.pallas import tpu as pltpu
from jax.experimental.pallas import tpu_sc as plsc

N = 10000
E = 320000
IN_CH = 128
HID = 64
G = 64

NC, NS = 2, 16
NW = NC * NS
CH = 128
SUP = 8
NPAD = 10240
EROWS = 2560
ROWS_PER_W = EROWS // NW
NSUP = ROWS_PER_W // SUP
STRIPE = NPAD // NS

def _worker_id():
    return lax.axis_index("s") * NC + lax.axis_index("c")


def _mesh():
    return plsc.VectorSubcoreMesh(
        core_axis_name="c", subcore_axis_name="s",
        num_cores=NC, num_subcores=NS)


@functools.cache
def _make_deg_kernel():
    return pl.kernel(
        _deg_body,
        out_type=jax.ShapeDtypeStruct((NC, NPAD), jnp.float32),
        mesh=_mesh(),
        scratch_types=[
            pltpu.VMEM_SHARED((NPAD,), jnp.float32),
            pltpu.VMEM((SUP, CH), jnp.int32),
            pltpu.VMEM((CH,), jnp.float32),
            pltpu.VMEM((STRIPE,), jnp.float32),
        ],
    )


def _deg_body(dst_hbm, out_hbm, deg_sh, dst_v, ones_v, zb_v):
    c = lax.axis_index("c")
    s = lax.axis_index("s")
    w = _worker_id()

    def fill(i, _):
        zb_v[pl.ds(i * 16, 16)] = jnp.zeros((16,), jnp.float32)
        return 0
    lax.fori_loop(0, STRIPE // 16, fill, 0)
    for j in range(CH // 16):
        ones_v[pl.ds(j * 16, 16)] = jnp.ones((16,), jnp.float32)
    pltpu.sync_copy(zb_v, deg_sh.at[pl.ds(s * STRIPE, STRIPE)])
    plsc.subcore_barrier()

    def step(i, _):
        base = w * ROWS_PER_W + i * SUP
        pltpu.sync_copy(dst_hbm.at[pl.ds(base, SUP)], dst_v)
        for j in range(SUP):
            pltpu.sync_copy(ones_v, deg_sh.at[dst_v.at[j]], add=True)
        return 0
    lax.fori_loop(0, NSUP, step, 0)

    plsc.subcore_barrier()
    @pl.when(s == 0)
    def _():
        pltpu.sync_copy(deg_sh, out_hbm.at[c])


GRP = 2
NGRP = ROWS_PER_W // GRP


@functools.cache
def _make_agg_kernel():
    return pl.kernel(
        _agg_body,
        out_type=jax.ShapeDtypeStruct((NC, NPAD, HID), jnp.float32),
        mesh=_mesh(),
        compiler_params=pltpu.CompilerParams(use_tc_tiling_on_sc=False),
        scratch_types=[
            pltpu.VMEM_SHARED((NPAD, HID), jnp.float32),
            pltpu.VMEM_SHARED((NPAD, HID), jnp.float32),
            pltpu.VMEM((3, GRP, CH), jnp.int32),
            pltpu.VMEM((3, GRP, CH), jnp.int32),
            pltpu.VMEM((2 * GRP, CH, HID), jnp.float32),
            pltpu.SemaphoreType.DMA,
            pltpu.SemaphoreType.DMA,
            pltpu.SemaphoreType.DMA,
        ],
    )


def _agg_body(hp_hbm, src_hbm, dst_hbm, out_hbm,
              acc_sh, hp_sh, src_v, dst_v, rows_v, sem_i, sem_g, sem_s):
    c = lax.axis_index("c")
    s = lax.axis_index("s")
    w = _worker_id()
    base = w * ROWS_PER_W

    def drain(sem, n, idx_shape=False):
        for _ in range(n):
            if idx_shape:
                pltpu.make_async_copy(
                    src_hbm.at[pl.ds(0, GRP)], src_v.at[0], sem).wait()
            else:
                pltpu.make_async_copy(
                    hp_hbm.at[pl.ds(0, CH)], rows_v.at[0], sem).wait()

    def load_idx(g, slot):
        pltpu.async_copy(src_hbm.at[pl.ds(base + g * GRP, GRP)],
                         src_v.at[slot], sem_i)
        pltpu.async_copy(dst_hbm.at[pl.ds(base + g * GRP, GRP)],
                         dst_v.at[slot], sem_i)

    def fire_gathers(g, rowset):
        slot = lax.rem(g, 3)
        for j in range(GRP):
            pltpu.async_copy(hp_sh.at[src_v.at[slot, j]],
                             rows_v.at[rowset * GRP + j], sem_g)

    load_idx(0, 0)

    def zfill(i, _):
        rows_v[0, pl.ds(i * 16, 16), 0:HID] = jnp.zeros((16, HID), jnp.float32)
        return 0
    lax.fori_loop(0, CH // 16, zfill, 0)
    for k in range(STRIPE // CH):
        pltpu.sync_copy(rows_v.at[0], acc_sh.at[pl.ds(s * STRIPE + k * CH, CH)])
    pltpu.sync_copy(hp_hbm.at[pl.ds(s * STRIPE, STRIPE)],
                    hp_sh.at[pl.ds(s * STRIPE, STRIPE)])
    plsc.subcore_barrier()

    drain(sem_i, 2, idx_shape=True)
    fire_gathers(0, 0)
    load_idx(1, 1)

    def grp_step(g, _):
        nxt = (g + 1) % 2
        @pl.when(g >= 1)
        def _():
            drain(sem_s, GRP)

        @pl.when(g <= NGRP - 2)
        def _():
            drain(sem_i, 2, idx_shape=True)
            fire_gathers(g + 1, nxt)

        @pl.when(g <= NGRP - 3)
        def _():
            load_idx(g + 2, lax.rem(g + 2, 3))

        drain(sem_g, GRP)
        cur = g % 2
        slot = lax.rem(g, 3)
        for j in range(GRP):
            pltpu.async_copy(rows_v.at[cur * GRP + j],
                             acc_sh.at[dst_v.at[slot, j]], sem_s,
                             add=True)
        return 0
    lax.fori_loop(0, NGRP, grp_step, 0)
    drain(sem_s, GRP)

    plsc.subcore_barrier()
    pltpu.sync_copy(acc_sh.at[pl.ds(s * STRIPE, STRIPE)],
                    out_hbm.at[c, pl.ds(s * STRIPE, STRIPE)])


def _dis(degp_ref):
    dsum = lax.dot_general(degp_ref[...], jnp.ones((2, 1), jnp.float32),
                           (((0,), (0,)), ((), ())),
                           preferred_element_type=jnp.float32)
    return lax.rsqrt(dsum + 1.0)


def _prep_body(x_ref, w1_ref, degp_ref, hp_ref):
    dis = _dis(degp_ref)
    hp_ref[0:N, :] = jnp.dot(x_ref[...], w1_ref[...],
                             preferred_element_type=jnp.float32) * dis[0:N]
    hp_ref[N:NPAD, :] = jnp.zeros((NPAD - N, HID), jnp.float32)


def _mid_body(a_ref, hp_ref, degp_ref, b1_ref, w2_ref, hp2_ref):
    dis = _dis(degp_ref)
    tot = a_ref[0] + a_ref[1] + hp_ref[...]
    h1 = jnp.maximum(tot * dis + b1_ref[...], 0.0)
    hp2_ref[...] = jnp.dot(h1, w2_ref[...],
                           preferred_element_type=jnp.float32) * dis


def _head_body(a_ref, hp_ref, degp_ref, b2_ref, batch_ref,
               wl1_ref, bl1_ref, wl2_ref, bl2_ref, out_ref):
    dis = _dis(degp_ref)
    tot = a_ref[0, 0:N, :] + a_ref[1, 0:N, :] + hp_ref[0:N, :]
    h2 = jnp.maximum(tot * dis[0:N] + b2_ref[...], 0.0)
    gid = lax.broadcasted_iota(jnp.int32, (N, G), 1)
    oh = (batch_ref[...] == gid).astype(jnp.float32)
    sums = lax.dot_general(oh, h2, (((0,), (0,)), ((), ())),
                           preferred_element_type=jnp.float32)
    counts = lax.dot_general(oh, jnp.ones((N, 1), jnp.float32),
                             (((0,), (0,)), ((), ())),
                             preferred_element_type=jnp.float32)
    pooled = sums / jnp.maximum(counts, 1.0)
    h = jnp.maximum(jnp.dot(pooled, wl1_ref[...],
                            preferred_element_type=jnp.float32) + bl1_ref[...], 0.0)
    logits = jnp.dot(h, wl2_ref[...],
                     preferred_element_type=jnp.float32) + bl2_ref[...]
    out_ref[...] = 1.0 / (1.0 + jnp.exp(-logits))


def kernel(x, edge_index, batch, W1, b1, W2, b2, Wl1, bl1, Wl2, bl2):
    pad = EROWS * CH - E
    fill = jnp.full((pad,), N, dtype=jnp.int32)
    src2d = jnp.concatenate([edge_index[0], fill]).reshape(EROWS, CH)
    dst2d = jnp.concatenate([edge_index[1], fill]).reshape(EROWS, CH)
    batch2d = batch.reshape(N, 1)

    degp = _make_deg_kernel()(dst2d)

    hp1 = pl.pallas_call(
        _prep_body,
        out_shape=jax.ShapeDtypeStruct((NPAD, HID), jnp.float32),
    )(x, W1, degp)

    a1 = _make_agg_kernel()(hp1, src2d, dst2d)

    hp2 = pl.pallas_call(
        _mid_body,
        out_shape=jax.ShapeDtypeStruct((NPAD, HID), jnp.float32),
    )(a1, hp1, degp, b1.reshape(1, HID), W2)

    a2 = _make_agg_kernel()(hp2, src2d, dst2d)

    out = pl.pallas_call(
        _head_body,
        out_shape=jax.ShapeDtypeStruct((G, 1), jnp.float32),
    )(a2, hp2, degp, b2.reshape(1, HID), batch2d,
      Wl1, bl1.reshape(1, HID), Wl2, bl2.reshape(1, 1))
    return out.reshape(-1)


# speedup vs baseline: 39.2059x; 1.0391x over previous
import functools

import jax
import jax.numpy as jnp
from jax import lax
from jax.experimental import pallas as pl
from jax.experimental.pallas import tpu as pltpu
from jax.experimental.pallas import tpu_sc as plsc

N = 10000
E = 320000
IN_CH = 128
HID = 64
G = 64

NC, NS = 2, 16
NW = NC * NS
CH = 128
SUP = 8
NPAD = 10240
EROWS = 2560
ROWS_PER_W = EROWS // NW
NSUP = ROWS_PER_W // SUP
STRIPE = NPAD // NS

def _worker_id():
    return lax.axis_index("s") * NC + lax.axis_index("c")


def _mesh():
    return plsc.VectorSubcoreMesh(
        core_axis_name="c", subcore_axis_name="s",
        num_cores=NC, num_subcores=NS)


@functools.cache
def _make_deg_kernel():
    return pl.kernel(
        _deg_body,
        out_type=jax.ShapeDtypeStruct((NC, NPAD), jnp.float32),
        mesh=_mesh(),
        scratch_types=[
            pltpu.VMEM_SHARED((NPAD,), jnp.float32),
            pltpu.VMEM((SUP, CH), jnp.int32),
            pltpu.VMEM((CH,), jnp.float32),
            pltpu.VMEM((STRIPE,), jnp.float32),
        ],
    )


def _deg_body(dst_hbm, out_hbm, deg_sh, dst_v, ones_v, zb_v):
    c = lax.axis_index("c")
    s = lax.axis_index("s")
    w = _worker_id()

    def fill(i, _):
        zb_v[pl.ds(i * 16, 16)] = jnp.zeros((16,), jnp.float32)
        return 0
    lax.fori_loop(0, STRIPE // 16, fill, 0)
    for j in range(CH // 16):
        ones_v[pl.ds(j * 16, 16)] = jnp.ones((16,), jnp.float32)
    pltpu.sync_copy(zb_v, deg_sh.at[pl.ds(s * STRIPE, STRIPE)])
    plsc.subcore_barrier()

    def step(i, _):
        base = w * ROWS_PER_W + i * SUP
        pltpu.sync_copy(dst_hbm.at[pl.ds(base, SUP)], dst_v)
        for j in range(SUP):
            pltpu.sync_copy(ones_v, deg_sh.at[dst_v.at[j]], add=True)
        return 0
    lax.fori_loop(0, NSUP, step, 0)

    plsc.subcore_barrier()
    @pl.when(s == 0)
    def _():
        pltpu.sync_copy(deg_sh, out_hbm.at[c])


GRP = 2
NGRP = ROWS_PER_W // GRP


@functools.cache
def _make_agg_kernel():
    return pl.kernel(
        _agg_body,
        out_type=jax.ShapeDtypeStruct((NC, NPAD, HID), jnp.float32),
        mesh=_mesh(),
        compiler_params=pltpu.CompilerParams(use_tc_tiling_on_sc=False),
        scratch_types=[
            pltpu.VMEM_SHARED((NPAD, HID), jnp.float32),
            pltpu.VMEM_SHARED((NPAD, HID), jnp.float32),
            pltpu.VMEM((3, GRP, CH), jnp.int32),
            pltpu.VMEM((3, GRP, CH), jnp.int32),
            pltpu.VMEM((2 * GRP, CH, HID), jnp.float32),
            pltpu.SemaphoreType.DMA,
            pltpu.SemaphoreType.DMA,
            pltpu.SemaphoreType.DMA,
        ],
    )


def _agg_body(hp_hbm, src_hbm, dst_hbm, out_hbm,
              acc_sh, hp_sh, src_v, dst_v, rows_v, sem_i, sem_g, sem_s):
    c = lax.axis_index("c")
    s = lax.axis_index("s")
    w = _worker_id()
    base = w * ROWS_PER_W

    def drain(sem, n, idx_shape=False):
        for _ in range(n):
            if idx_shape:
                pltpu.make_async_copy(
                    src_hbm.at[pl.ds(0, GRP)], src_v.at[0], sem).wait()
            else:
                pltpu.make_async_copy(
                    hp_hbm.at[pl.ds(0, CH)], rows_v.at[0], sem).wait()

    def load_idx(g, slot):
        pltpu.async_copy(src_hbm.at[pl.ds(base + g * GRP, GRP)],
                         src_v.at[slot], sem_i)
        pltpu.async_copy(dst_hbm.at[pl.ds(base + g * GRP, GRP)],
                         dst_v.at[slot], sem_i)

    def fire_gathers(g, rowset):
        slot = lax.rem(g, 3)
        for j in range(GRP):
            pltpu.async_copy(hp_sh.at[src_v.at[slot, j]],
                             rows_v.at[rowset * GRP + j], sem_g)

    load_idx(0, 0)

    def zfill(i, _):
        rows_v[0, pl.ds(i * 16, 16), 0:HID] = jnp.zeros((16, HID), jnp.float32)
        return 0
    lax.fori_loop(0, CH // 16, zfill, 0)
    for k in range(STRIPE // CH):
        pltpu.sync_copy(rows_v.at[0], acc_sh.at[pl.ds(s * STRIPE + k * CH, CH)])
    pltpu.sync_copy(hp_hbm.at[pl.ds(s * STRIPE, STRIPE)],
                    hp_sh.at[pl.ds(s * STRIPE, STRIPE)])
    plsc.subcore_barrier()

    drain(sem_i, 2, idx_shape=True)
    fire_gathers(0, 0)
    load_idx(1, 1)

    def grp_step(g, _):
        nxt = (g + 1) % 2
        @pl.when(g >= 1)
        def _():
            drain(sem_s, GRP)

        @pl.when(g <= NGRP - 2)
        def _():
            drain(sem_i, 2, idx_shape=True)
            fire_gathers(g + 1, nxt)

        @pl.when(g <= NGRP - 3)
        def _():
            load_idx(g + 2, lax.rem(g + 2, 3))

        drain(sem_g, GRP)
        cur = g % 2
        slot = lax.rem(g, 3)
        for j in range(GRP):
            pltpu.async_copy(rows_v.at[cur * GRP + j],
                             acc_sh.at[dst_v.at[slot, j]], sem_s,
                             add=True)
        return 0
    lax.fori_loop(0, NGRP, grp_step, 0)
    drain(sem_s, GRP)

    plsc.subcore_barrier()
    pltpu.sync_copy(acc_sh.at[pl.ds(s * STRIPE, STRIPE)],
                    out_hbm.at[c, pl.ds(s * STRIPE, STRIPE)])


def _dis(degp_ref):
    dsum = lax.dot_general(degp_ref[...], jnp.ones((2, 1), jnp.float32),
                           (((0,), (0,)), ((), ())),
                           preferred_element_type=jnp.float32)
    return lax.rsqrt(dsum + 1.0)


def _prep_body(x_ref, w1_ref, degp_ref, hp_ref):
    dis = _dis(degp_ref)
    hp_ref[0:N, :] = jnp.dot(x_ref[...], w1_ref[...],
                             preferred_element_type=jnp.float32) * dis[0:N]
    hp_ref[N:NPAD, :] = jnp.zeros((NPAD - N, HID), jnp.float32)


def _mid_body(a_ref, hp_ref, degp_ref, b1_ref, w2_ref, hp2_ref):
    dis = _dis(degp_ref)
    tot = a_ref[0] + a_ref[1] + hp_ref[...]
    h1 = jnp.maximum(tot * dis + b1_ref[...], 0.0)
    hp2_ref[...] = jnp.dot(h1, w2_ref[...],
                           preferred_element_type=jnp.float32) * dis


def _head_body(a_ref, hp_ref, degp_ref, b2_ref, batch_ref,
               wl1_ref, bl1_ref, wl2_ref, bl2_ref, out_ref):
    dis = _dis(degp_ref)
    tot = a_ref[0, 0:N, :] + a_ref[1, 0:N, :] + hp_ref[0:N, :]
    h2 = jnp.maximum(tot * dis[0:N] + b2_ref[...], 0.0)
    gid = lax.broadcasted_iota(jnp.int32, (N, G), 1)
    oh = (batch_ref[...] == gid).astype(jnp.float32)
    sums = lax.dot_general(oh, h2, (((0,), (0,)), ((), ())),
                           preferred_element_type=jnp.float32)
    counts = lax.dot_general(oh, jnp.ones((N, 1), jnp.float32),
                             (((0,), (0,)), ((), ())),
                             preferred_element_type=jnp.float32)
    pooled = sums / jnp.maximum(counts, 1.0)
    h = jnp.maximum(jnp.dot(pooled, wl1_ref[...],
                            preferred_element_type=jnp.float32) + bl1_ref[...], 0.0)
    logits = jnp.dot(h, wl2_ref[...],
                     preferred_element_type=jnp.float32) + bl2_ref[...]
    out_ref[...] = 1.0 / (1.0 + jnp.exp(-logits))


def kernel(x, edge_index, batch, W1, b1, W2, b2, Wl1, bl1, Wl2, bl2):
    pad = EROWS * CH - E
    ei_pad = jnp.pad(edge_index, ((0, 0), (0, pad)), constant_values=N)
    ei_pad = ei_pad.reshape(2, EROWS, CH)
    src2d = ei_pad[0]
    dst2d = ei_pad[1]
    batch2d = batch.reshape(N, 1)

    degp = _make_deg_kernel()(dst2d)

    hp1 = pl.pallas_call(
        _prep_body,
        out_shape=jax.ShapeDtypeStruct((NPAD, HID), jnp.float32),
    )(x, W1, degp)

    a1 = _make_agg_kernel()(hp1, src2d, dst2d)

    hp2 = pl.pallas_call(
        _mid_body,
        out_shape=jax.ShapeDtypeStruct((NPAD, HID), jnp.float32),
    )(a1, hp1, degp, b1.reshape(1, HID), W2)

    a2 = _make_agg_kernel()(hp2, src2d, dst2d)

    out = pl.pallas_call(
        _head_body,
        out_shape=jax.ShapeDtypeStruct((G, 1), jnp.float32),
    )(a2, hp2, degp, b2.reshape(1, HID), batch2d,
      Wl1, bl1.reshape(1, HID), Wl2, bl2.reshape(1, 1))
    return out.reshape(-1)


# speedup vs baseline: 41.3728x; 1.0553x over previous
import functools

import jax
import jax.numpy as jnp
from jax import lax
from jax.experimental import pallas as pl
from jax.experimental.pallas import tpu as pltpu
from jax.experimental.pallas import tpu_sc as plsc

N = 10000
E = 320000
IN_CH = 128
HID = 64
G = 64

NC, NS = 2, 16
NW = NC * NS
CH = 128
SUP = 8
NPAD = 10240
EROWS = 2560
ROWS_PER_W = EROWS // NW
NSUP = ROWS_PER_W // SUP
STRIPE = NPAD // NS

def _worker_id():
    return lax.axis_index("s") * NC + lax.axis_index("c")


def _mesh():
    return plsc.VectorSubcoreMesh(
        core_axis_name="c", subcore_axis_name="s",
        num_cores=NC, num_subcores=NS)


@functools.cache
def _make_deg_kernel():
    return pl.kernel(
        _deg_body,
        out_type=jax.ShapeDtypeStruct((NC, NPAD), jnp.float32),
        mesh=_mesh(),
        scratch_types=[
            pltpu.VMEM_SHARED((NPAD,), jnp.float32),
            pltpu.VMEM((SUP, CH), jnp.int32),
            pltpu.VMEM((CH,), jnp.float32),
            pltpu.VMEM((STRIPE,), jnp.float32),
        ],
    )


def _deg_body(dst_hbm, out_hbm, deg_sh, dst_v, ones_v, zb_v):
    c = lax.axis_index("c")
    s = lax.axis_index("s")
    w = _worker_id()

    def fill(i, _):
        zb_v[pl.ds(i * 16, 16)] = jnp.zeros((16,), jnp.float32)
        return 0
    lax.fori_loop(0, STRIPE // 16, fill, 0)
    for j in range(CH // 16):
        ones_v[pl.ds(j * 16, 16)] = jnp.ones((16,), jnp.float32)
    pltpu.sync_copy(zb_v, deg_sh.at[pl.ds(s * STRIPE, STRIPE)])
    plsc.subcore_barrier()

    def step(i, _):
        base = w * ROWS_PER_W + i * SUP
        pltpu.sync_copy(dst_hbm.at[pl.ds(base, SUP)], dst_v)
        for j in range(SUP):
            pltpu.sync_copy(ones_v, deg_sh.at[dst_v.at[j]], add=True)
        return 0
    lax.fori_loop(0, NSUP, step, 0)

    plsc.subcore_barrier()
    @pl.when(s == 0)
    def _():
        pltpu.sync_copy(deg_sh, out_hbm.at[c])


HHID = HID // NC
GRP = 4
ROWS_PER_T = EROWS // NS
NGRP = ROWS_PER_T // GRP


@functools.cache
def _make_agg_kernel():
    return pl.kernel(
        _agg_body,
        out_type=jax.ShapeDtypeStruct((NPAD, HID), jnp.float32),
        mesh=_mesh(),
        compiler_params=pltpu.CompilerParams(use_tc_tiling_on_sc=False),
        scratch_types=[
            pltpu.VMEM_SHARED((NPAD, HHID), jnp.float32),
            pltpu.VMEM_SHARED((NPAD, HHID), jnp.float32),
            pltpu.VMEM((3, GRP, CH), jnp.int32),
            pltpu.VMEM((3, GRP, CH), jnp.int32),
            pltpu.VMEM((2 * GRP, CH, HHID), jnp.float32),
            pltpu.SemaphoreType.DMA,
            pltpu.SemaphoreType.DMA,
            pltpu.SemaphoreType.DMA,
        ],
    )


def _agg_body(hp_hbm, src_hbm, dst_hbm, out_hbm,
              acc_sh, hp_sh, src_v, dst_v, rows_v, sem_i, sem_g, sem_s):
    c = lax.axis_index("c")
    s = lax.axis_index("s")
    base = s * ROWS_PER_T

    def drain(sem, n, idx_shape=False):
        for _ in range(n):
            if idx_shape:
                pltpu.make_async_copy(
                    src_hbm.at[pl.ds(0, GRP)], src_v.at[0], sem).wait()
            else:
                pltpu.make_async_copy(
                    hp_hbm.at[pl.ds(0, CH), pl.ds(0, HHID)],
                    rows_v.at[0], sem).wait()

    def load_idx(g, slot):
        pltpu.async_copy(src_hbm.at[pl.ds(base + g * GRP, GRP)],
                         src_v.at[slot], sem_i)
        pltpu.async_copy(dst_hbm.at[pl.ds(base + g * GRP, GRP)],
                         dst_v.at[slot], sem_i)

    def fire_gathers(g, rowset):
        slot = lax.rem(g, 3)
        for j in range(GRP):
            pltpu.async_copy(hp_sh.at[src_v.at[slot, j]],
                             rows_v.at[rowset * GRP + j], sem_g)

    load_idx(0, 0)

    def zfill(i, _):
        rows_v[0, pl.ds(i * 16, 16), 0:HHID] = jnp.zeros((16, HHID), jnp.float32)
        return 0
    lax.fori_loop(0, CH // 16, zfill, 0)
    for k in range(STRIPE // CH):
        pltpu.sync_copy(rows_v.at[0], acc_sh.at[pl.ds(s * STRIPE + k * CH, CH)])
    pltpu.sync_copy(hp_hbm.at[pl.ds(s * STRIPE, STRIPE), pl.ds(c * HHID, HHID)],
                    hp_sh.at[pl.ds(s * STRIPE, STRIPE)])
    plsc.subcore_barrier()

    drain(sem_i, 2, idx_shape=True)
    fire_gathers(0, 0)
    load_idx(1, 1)

    def grp_step(g, _):
        nxt = (g + 1) % 2
        @pl.when(g >= 1)
        def _():
            drain(sem_s, GRP)

        @pl.when(g <= NGRP - 2)
        def _():
            drain(sem_i, 2, idx_shape=True)
            fire_gathers(g + 1, nxt)

        @pl.when(g <= NGRP - 3)
        def _():
            load_idx(g + 2, lax.rem(g + 2, 3))

        drain(sem_g, GRP)
        cur = g % 2
        slot = lax.rem(g, 3)
        for j in range(GRP):
            pltpu.async_copy(rows_v.at[cur * GRP + j],
                             acc_sh.at[dst_v.at[slot, j]], sem_s,
                             add=True)
        return 0
    lax.fori_loop(0, NGRP, grp_step, 0)
    drain(sem_s, GRP)

    plsc.subcore_barrier()
    pltpu.sync_copy(acc_sh.at[pl.ds(s * STRIPE, STRIPE)],
                    out_hbm.at[pl.ds(s * STRIPE, STRIPE), pl.ds(c * HHID, HHID)])


def _dis(degp_ref):
    dsum = lax.dot_general(degp_ref[...], jnp.ones((2, 1), jnp.float32),
                           (((0,), (0,)), ((), ())),
                           preferred_element_type=jnp.float32)
    return lax.rsqrt(dsum + 1.0)


def _prep_body(x_ref, w1_ref, degp_ref, hp_ref):
    dis = _dis(degp_ref)
    hp_ref[0:N, :] = jnp.dot(x_ref[...], w1_ref[...],
                             preferred_element_type=jnp.float32) * dis[0:N]
    hp_ref[N:NPAD, :] = jnp.zeros((NPAD - N, HID), jnp.float32)


def _mid_body(a_ref, hp_ref, degp_ref, b1_ref, w2_ref, hp2_ref):
    dis = _dis(degp_ref)
    tot = a_ref[...] + hp_ref[...]
    h1 = jnp.maximum(tot * dis + b1_ref[...], 0.0)
    hp2_ref[...] = jnp.dot(h1, w2_ref[...],
                           preferred_element_type=jnp.float32) * dis


def _head_body(a_ref, hp_ref, degp_ref, b2_ref, batch_ref,
               wl1_ref, bl1_ref, wl2_ref, bl2_ref, out_ref):
    dis = _dis(degp_ref)
    tot = a_ref[0:N, :] + hp_ref[0:N, :]
    h2 = jnp.maximum(tot * dis[0:N] + b2_ref[...], 0.0)
    gid = lax.broadcasted_iota(jnp.int32, (N, G), 1)
    oh = (batch_ref[...] == gid).astype(jnp.float32)
    sums = lax.dot_general(oh, h2, (((0,), (0,)), ((), ())),
                           preferred_element_type=jnp.float32)
    counts = lax.dot_general(oh, jnp.ones((N, 1), jnp.float32),
                             (((0,), (0,)), ((), ())),
                             preferred_element_type=jnp.float32)
    pooled = sums / jnp.maximum(counts, 1.0)
    h = jnp.maximum(jnp.dot(pooled, wl1_ref[...],
                            preferred_element_type=jnp.float32) + bl1_ref[...], 0.0)
    logits = jnp.dot(h, wl2_ref[...],
                     preferred_element_type=jnp.float32) + bl2_ref[...]
    out_ref[...] = 1.0 / (1.0 + jnp.exp(-logits))


def kernel(x, edge_index, batch, W1, b1, W2, b2, Wl1, bl1, Wl2, bl2):
    pad = EROWS * CH - E
    ei_pad = jnp.pad(edge_index, ((0, 0), (0, pad)), constant_values=N)
    ei_pad = ei_pad.reshape(2, EROWS, CH)
    src2d = ei_pad[0]
    dst2d = ei_pad[1]
    batch2d = batch.reshape(N, 1)

    degp = _make_deg_kernel()(dst2d)

    hp1 = pl.pallas_call(
        _prep_body,
        out_shape=jax.ShapeDtypeStruct((NPAD, HID), jnp.float32),
    )(x, W1, degp)

    a1 = _make_agg_kernel()(hp1, src2d, dst2d)

    hp2 = pl.pallas_call(
        _mid_body,
        out_shape=jax.ShapeDtypeStruct((NPAD, HID), jnp.float32),
    )(a1, hp1, degp, b1.reshape(1, HID), W2)

    a2 = _make_agg_kernel()(hp2, src2d, dst2d)

    out = pl.pallas_call(
        _head_body,
        out_shape=jax.ShapeDtypeStruct((G, 1), jnp.float32),
    )(a2, hp2, degp, b2.reshape(1, HID), batch2d,
      Wl1, bl1.reshape(1, HID), Wl2, bl2.reshape(1, 1))
    return out.reshape(-1)


# speedup vs baseline: 43.7284x; 1.0569x over previous
import functools

import jax
import jax.numpy as jnp
from jax import lax
from jax.experimental import pallas as pl
from jax.experimental.pallas import tpu as pltpu
from jax.experimental.pallas import tpu_sc as plsc

N = 10000
E = 320000
IN_CH = 128
HID = 64
G = 64

NC, NS = 2, 16
NW = NC * NS
CH = 128
SUP = 8
NPAD = 10240
EROWS = 2560
ROWS_PER_W = EROWS // NW
NSUP = ROWS_PER_W // SUP
STRIPE = NPAD // NS

def _worker_id():
    return lax.axis_index("s") * NC + lax.axis_index("c")


def _mesh():
    return plsc.VectorSubcoreMesh(
        core_axis_name="c", subcore_axis_name="s",
        num_cores=NC, num_subcores=NS)


DGRP = 4
DNGRP = ROWS_PER_W // DGRP


@functools.cache
def _make_deg_kernel():
    return pl.kernel(
        _deg_body,
        out_type=jax.ShapeDtypeStruct((NC, NPAD), jnp.float32),
        mesh=_mesh(),
        scratch_types=[
            pltpu.VMEM_SHARED((NPAD,), jnp.float32),
            pltpu.VMEM((3, DGRP, CH), jnp.int32),
            pltpu.VMEM((CH,), jnp.float32),
            pltpu.VMEM((STRIPE,), jnp.float32),
            pltpu.SemaphoreType.DMA,
            pltpu.SemaphoreType.DMA,
        ],
    )


def _deg_body(dst_hbm, out_hbm, deg_sh, dst_v, ones_v, zb_v, sem_i, sem_s):
    c = lax.axis_index("c")
    s = lax.axis_index("s")
    w = _worker_id()
    base = w * ROWS_PER_W

    def drain(sem, n, idx_shape=False):
        for _ in range(n):
            if idx_shape:
                pltpu.make_async_copy(
                    dst_hbm.at[pl.ds(0, DGRP)], dst_v.at[0], sem).wait()
            else:
                pltpu.make_async_copy(
                    out_hbm.at[0, pl.ds(0, CH)], ones_v, sem).wait()

    def load_idx(g, slot):
        pltpu.async_copy(dst_hbm.at[pl.ds(base + g * DGRP, DGRP)],
                         dst_v.at[slot], sem_i)

    load_idx(0, 0)

    def fill(i, _):
        zb_v[pl.ds(i * 16, 16)] = jnp.zeros((16,), jnp.float32)
        return 0
    lax.fori_loop(0, STRIPE // 16, fill, 0)
    for j in range(CH // 16):
        ones_v[pl.ds(j * 16, 16)] = jnp.ones((16,), jnp.float32)
    pltpu.sync_copy(zb_v, deg_sh.at[pl.ds(s * STRIPE, STRIPE)])
    plsc.subcore_barrier()
    load_idx(1, 1)

    def grp_step(g, _):
        @pl.when(g >= 1)
        def _():
            drain(sem_s, DGRP)
        drain(sem_i, 1, idx_shape=True)

        @pl.when(g <= DNGRP - 3)
        def _():
            load_idx(g + 2, lax.rem(g + 2, 3))
        slot = lax.rem(g, 3)
        for j in range(DGRP):
            pltpu.async_copy(ones_v, deg_sh.at[dst_v.at[slot, j]], sem_s,
                             add=True)
        return 0
    lax.fori_loop(0, DNGRP, grp_step, 0)
    drain(sem_s, DGRP)

    plsc.subcore_barrier()
    @pl.when(s == 0)
    def _():
        pltpu.sync_copy(deg_sh, out_hbm.at[c])


HHID = HID // NC
GRP = 8
ROWS_PER_T = EROWS // NS
NGRP = ROWS_PER_T // GRP


@functools.cache
def _make_agg_kernel():
    return pl.kernel(
        _agg_body,
        out_type=jax.ShapeDtypeStruct((NPAD, HID), jnp.float32),
        mesh=_mesh(),
        compiler_params=pltpu.CompilerParams(use_tc_tiling_on_sc=False),
        scratch_types=[
            pltpu.VMEM_SHARED((NPAD, HHID), jnp.float32),
            pltpu.VMEM_SHARED((NPAD, HHID), jnp.float32),
            pltpu.VMEM((3, GRP, CH), jnp.int32),
            pltpu.VMEM((3, GRP, CH), jnp.int32),
            pltpu.VMEM((2 * GRP, CH, HHID), jnp.float32),
            pltpu.SemaphoreType.DMA,
            pltpu.SemaphoreType.DMA,
            pltpu.SemaphoreType.DMA,
        ],
    )


def _agg_body(hp_hbm, src_hbm, dst_hbm, out_hbm,
              acc_sh, hp_sh, src_v, dst_v, rows_v, sem_i, sem_g, sem_s):
    c = lax.axis_index("c")
    s = lax.axis_index("s")
    base = s * ROWS_PER_T

    def drain(sem, n, idx_shape=False):
        for _ in range(n):
            if idx_shape:
                pltpu.make_async_copy(
                    src_hbm.at[pl.ds(0, GRP)], src_v.at[0], sem).wait()
            else:
                pltpu.make_async_copy(
                    hp_hbm.at[pl.ds(0, CH), pl.ds(0, HHID)],
                    rows_v.at[0], sem).wait()

    def load_idx(g, slot):
        pltpu.async_copy(src_hbm.at[pl.ds(base + g * GRP, GRP)],
                         src_v.at[slot], sem_i)
        pltpu.async_copy(dst_hbm.at[pl.ds(base + g * GRP, GRP)],
                         dst_v.at[slot], sem_i)

    def fire_gathers(g, rowset):
        slot = lax.rem(g, 3)
        for j in range(GRP):
            pltpu.async_copy(hp_sh.at[src_v.at[slot, j]],
                             rows_v.at[rowset * GRP + j], sem_g)

    load_idx(0, 0)

    def zfill(i, _):
        rows_v[0, pl.ds(i * 16, 16), 0:HHID] = jnp.zeros((16, HHID), jnp.float32)
        return 0
    lax.fori_loop(0, CH // 16, zfill, 0)
    for k in range(STRIPE // CH):
        pltpu.sync_copy(rows_v.at[0], acc_sh.at[pl.ds(s * STRIPE + k * CH, CH)])
    pltpu.sync_copy(hp_hbm.at[pl.ds(s * STRIPE, STRIPE), pl.ds(c * HHID, HHID)],
                    hp_sh.at[pl.ds(s * STRIPE, STRIPE)])
    plsc.subcore_barrier()

    drain(sem_i, 2, idx_shape=True)
    fire_gathers(0, 0)
    load_idx(1, 1)

    def grp_step(g, _):
        nxt = (g + 1) % 2
        @pl.when(g >= 1)
        def _():
            drain(sem_s, GRP)

        @pl.when(g <= NGRP - 2)
        def _():
            drain(sem_i, 2, idx_shape=True)
            fire_gathers(g + 1, nxt)

        @pl.when(g <= NGRP - 3)
        def _():
            load_idx(g + 2, lax.rem(g + 2, 3))

        drain(sem_g, GRP)
        cur = g % 2
        slot = lax.rem(g, 3)
        for j in range(GRP):
            pltpu.async_copy(rows_v.at[cur * GRP + j],
                             acc_sh.at[dst_v.at[slot, j]], sem_s,
                             add=True)
        return 0
    lax.fori_loop(0, NGRP, grp_step, 0)
    drain(sem_s, GRP)

    plsc.subcore_barrier()
    pltpu.sync_copy(acc_sh.at[pl.ds(s * STRIPE, STRIPE)],
                    out_hbm.at[pl.ds(s * STRIPE, STRIPE), pl.ds(c * HHID, HHID)])


def _dis(degp_ref):
    dsum = lax.dot_general(degp_ref[...], jnp.ones((2, 1), jnp.float32),
                           (((0,), (0,)), ((), ())),
                           preferred_element_type=jnp.float32)
    return lax.rsqrt(dsum + 1.0)


def _prep_body(x_ref, w1_ref, degp_ref, hp_ref):
    dis = _dis(degp_ref)
    hp_ref[0:N, :] = jnp.dot(x_ref[...], w1_ref[...],
                             preferred_element_type=jnp.float32) * dis[0:N]
    hp_ref[N:NPAD, :] = jnp.zeros((NPAD - N, HID), jnp.float32)


def _mid_body(a_ref, hp_ref, degp_ref, b1_ref, w2_ref, hp2_ref):
    dis = _dis(degp_ref)
    tot = a_ref[...] + hp_ref[...]
    h1 = jnp.maximum(tot * dis + b1_ref[...], 0.0)
    hp2_ref[...] = jnp.dot(h1, w2_ref[...],
                           preferred_element_type=jnp.float32) * dis


def _head_body(a_ref, hp_ref, degp_ref, b2_ref, batch_ref,
               wl1_ref, bl1_ref, wl2_ref, bl2_ref, out_ref):
    dis = _dis(degp_ref)
    tot = a_ref[0:N, :] + hp_ref[0:N, :]
    h2 = jnp.maximum(tot * dis[0:N] + b2_ref[...], 0.0)
    gid = lax.broadcasted_iota(jnp.int32, (N, G), 1)
    oh = (batch_ref[...] == gid).astype(jnp.float32)
    sums = lax.dot_general(oh, h2, (((0,), (0,)), ((), ())),
                           preferred_element_type=jnp.float32)
    counts = lax.dot_general(oh, jnp.ones((N, 1), jnp.float32),
                             (((0,), (0,)), ((), ())),
                             preferred_element_type=jnp.float32)
    pooled = sums / jnp.maximum(counts, 1.0)
    h = jnp.maximum(jnp.dot(pooled, wl1_ref[...],
                            preferred_element_type=jnp.float32) + bl1_ref[...], 0.0)
    logits = jnp.dot(h, wl2_ref[...],
                     preferred_element_type=jnp.float32) + bl2_ref[...]
    out_ref[...] = 1.0 / (1.0 + jnp.exp(-logits))


def kernel(x, edge_index, batch, W1, b1, W2, b2, Wl1, bl1, Wl2, bl2):
    pad = EROWS * CH - E
    ei_pad = jnp.pad(edge_index, ((0, 0), (0, pad)), constant_values=N)
    ei_pad = ei_pad.reshape(2, EROWS, CH)
    src2d = ei_pad[0]
    dst2d = ei_pad[1]
    batch2d = batch.reshape(N, 1)

    degp = _make_deg_kernel()(dst2d)

    hp1 = pl.pallas_call(
        _prep_body,
        out_shape=jax.ShapeDtypeStruct((NPAD, HID), jnp.float32),
    )(x, W1, degp)

    a1 = _make_agg_kernel()(hp1, src2d, dst2d)

    hp2 = pl.pallas_call(
        _mid_body,
        out_shape=jax.ShapeDtypeStruct((NPAD, HID), jnp.float32),
    )(a1, hp1, degp, b1.reshape(1, HID), W2)

    a2 = _make_agg_kernel()(hp2, src2d, dst2d)

    out = pl.pallas_call(
        _head_body,
        out_shape=jax.ShapeDtypeStruct((G, 1), jnp.float32),
    )(a2, hp2, degp, b2.reshape(1, HID), batch2d,
      Wl1, bl1.reshape(1, HID), Wl2, bl2.reshape(1, 1))
    return out.reshape(-1)


# speedup vs baseline: 45.7186x; 1.0455x over previous
import functools

import jax
import jax.numpy as jnp
from jax import lax
from jax.experimental import pallas as pl
from jax.experimental.pallas import tpu as pltpu
from jax.experimental.pallas import tpu_sc as plsc

N = 10000
E = 320000
IN_CH = 128
HID = 64
G = 64

NC, NS = 2, 16
NW = NC * NS
CH = 128
SUP = 8
NPAD = 10240
EROWS = 2560
ROWS_PER_W = EROWS // NW
NSUP = ROWS_PER_W // SUP
STRIPE = NPAD // NS

def _worker_id():
    return lax.axis_index("s") * NC + lax.axis_index("c")


def _mesh():
    return plsc.VectorSubcoreMesh(
        core_axis_name="c", subcore_axis_name="s",
        num_cores=NC, num_subcores=NS)


DGRP = 4
DNGRP = ROWS_PER_W // DGRP


@functools.cache
def _make_deg_kernel():
    return pl.kernel(
        _deg_body,
        out_type=jax.ShapeDtypeStruct((NC, NPAD), jnp.float32),
        mesh=_mesh(),
        scratch_types=[
            pltpu.VMEM_SHARED((NPAD,), jnp.float32),
            pltpu.VMEM((3, DGRP, CH), jnp.int32),
            pltpu.VMEM((CH,), jnp.float32),
            pltpu.VMEM((STRIPE,), jnp.float32),
            pltpu.SemaphoreType.DMA,
            pltpu.SemaphoreType.DMA,
        ],
    )


def _deg_body(dst_hbm, out_hbm, deg_sh, dst_v, ones_v, zb_v, sem_i, sem_s):
    c = lax.axis_index("c")
    s = lax.axis_index("s")
    w = _worker_id()
    base = w * ROWS_PER_W

    def drain(sem, n, idx_shape=False):
        for _ in range(n):
            if idx_shape:
                pltpu.make_async_copy(
                    dst_hbm.at[pl.ds(0, DGRP)], dst_v.at[0], sem).wait()
            else:
                pltpu.make_async_copy(
                    out_hbm.at[0, pl.ds(0, CH)], ones_v, sem).wait()

    def load_idx(g, slot):
        pltpu.async_copy(dst_hbm.at[pl.ds(base + g * DGRP, DGRP)],
                         dst_v.at[slot], sem_i)

    load_idx(0, 0)

    def fill(i, _):
        zb_v[pl.ds(i * 16, 16)] = jnp.zeros((16,), jnp.float32)
        return 0
    lax.fori_loop(0, STRIPE // 16, fill, 0)
    for j in range(CH // 16):
        ones_v[pl.ds(j * 16, 16)] = jnp.ones((16,), jnp.float32)
    pltpu.sync_copy(zb_v, deg_sh.at[pl.ds(s * STRIPE, STRIPE)])
    plsc.subcore_barrier()
    load_idx(1, 1)

    def grp_step(g, _):
        @pl.when(g >= 1)
        def _():
            drain(sem_s, DGRP)
        drain(sem_i, 1, idx_shape=True)

        @pl.when(g <= DNGRP - 3)
        def _():
            load_idx(g + 2, lax.rem(g + 2, 3))
        slot = lax.rem(g, 3)
        for j in range(DGRP):
            pltpu.async_copy(ones_v, deg_sh.at[dst_v.at[slot, j]], sem_s,
                             add=True)
        return 0
    lax.fori_loop(0, DNGRP, grp_step, 0)
    drain(sem_s, DGRP)

    plsc.subcore_barrier()
    @pl.when(s == 0)
    def _():
        pltpu.sync_copy(deg_sh, out_hbm.at[c])


HHID = HID // NC
GRP = 8
ROWS_PER_T = EROWS // NS
NGRP = ROWS_PER_T // GRP


@functools.cache
def _make_agg_kernel():
    return pl.kernel(
        _agg_body,
        out_type=jax.ShapeDtypeStruct((NPAD, HID), jnp.float32),
        mesh=_mesh(),
        compiler_params=pltpu.CompilerParams(use_tc_tiling_on_sc=False),
        scratch_types=[
            pltpu.VMEM_SHARED((NPAD, HHID), jnp.float32),
            pltpu.VMEM_SHARED((NPAD, HHID), jnp.float32),
            pltpu.VMEM((3, GRP, CH), jnp.int32),
            pltpu.VMEM((3, GRP, CH), jnp.int32),
            pltpu.VMEM((2 * GRP, CH, HHID), jnp.float32),
            pltpu.SemaphoreType.DMA,
            pltpu.SemaphoreType.DMA,
            pltpu.SemaphoreType.DMA,
        ],
    )


def _agg_body(hp_hbm, src_hbm, dst_hbm, out_hbm,
              acc_sh, hp_sh, src_v, dst_v, rows_v, sem_i, sem_g, sem_s):
    c = lax.axis_index("c")
    s = lax.axis_index("s")
    base = s * ROWS_PER_T

    def drain(sem, n, idx_shape=False):
        for _ in range(n):
            if idx_shape:
                pltpu.make_async_copy(
                    src_hbm.at[pl.ds(0, GRP)], src_v.at[0], sem).wait()
            else:
                pltpu.make_async_copy(
                    hp_hbm.at[pl.ds(0, CH), pl.ds(0, HHID)],
                    rows_v.at[0], sem).wait()

    def load_idx(g, slot):
        pltpu.async_copy(src_hbm.at[pl.ds(base + g * GRP, GRP)],
                         src_v.at[slot], sem_i)
        pltpu.async_copy(dst_hbm.at[pl.ds(base + g * GRP, GRP)],
                         dst_v.at[slot], sem_i)

    def fire_gathers(g, rowset):
        slot = lax.rem(g, 3)
        for j in range(GRP):
            pltpu.async_copy(hp_sh.at[src_v.at[slot, j]],
                             rows_v.at[rowset * GRP + j], sem_g)

    load_idx(0, 0)

    def zfill(i, _):
        rows_v[0, pl.ds(i * 16, 16), 0:HHID] = jnp.zeros((16, HHID), jnp.float32)
        return 0
    lax.fori_loop(0, CH // 16, zfill, 0)
    for k in range(STRIPE // CH):
        pltpu.sync_copy(rows_v.at[0], acc_sh.at[pl.ds(s * STRIPE + k * CH, CH)])
    pltpu.sync_copy(hp_hbm.at[pl.ds(s * STRIPE, STRIPE), pl.ds(c * HHID, HHID)],
                    hp_sh.at[pl.ds(s * STRIPE, STRIPE)])
    plsc.subcore_barrier()

    drain(sem_i, 2, idx_shape=True)
    fire_gathers(0, 0)
    load_idx(1, 1)

    def grp_step(g, _):
        nxt = (g + 1) % 2
        @pl.when(g >= 1)
        def _():
            drain(sem_s, GRP)

        @pl.when(g <= NGRP - 2)
        def _():
            drain(sem_i, 2, idx_shape=True)
            fire_gathers(g + 1, nxt)

        @pl.when(g <= NGRP - 3)
        def _():
            load_idx(g + 2, lax.rem(g + 2, 3))

        drain(sem_g, GRP)
        cur = g % 2
        slot = lax.rem(g, 3)
        for j in range(GRP):
            pltpu.async_copy(rows_v.at[cur * GRP + j],
                             acc_sh.at[dst_v.at[slot, j]], sem_s,
                             add=True)
        return 0
    lax.fori_loop(0, NGRP, grp_step, 0)
    drain(sem_s, GRP)

    plsc.subcore_barrier()
    pltpu.sync_copy(acc_sh.at[pl.ds(s * STRIPE, STRIPE)],
                    out_hbm.at[pl.ds(s * STRIPE, STRIPE), pl.ds(c * HHID, HHID)])


NP2 = NPAD // 2
N2 = N // 2


def _dis128(degp_ref):
    dsum = lax.dot_general(degp_ref[...], jnp.ones((2, 1), jnp.float32),
                           (((0,), (0,)), ((), ())),
                           preferred_element_type=jnp.float32)
    dis = lax.rsqrt(dsum + 1.0)
    d2 = dis.reshape(NP2, 2)
    lane = lax.broadcasted_iota(jnp.int32, (2, 2 * HID), 1)
    rowi = lax.broadcasted_iota(jnp.int32, (2, 2 * HID), 0)
    sel = jnp.where((lane // HID) == rowi, 1.0, 0.0)
    return lax.dot_general(d2, sel, (((1,), (0,)), ((), ())),
                           preferred_element_type=jnp.float32)


def _blockdiag(w_ref):
    w = w_ref[...]
    kin = w.shape[0]
    z = jnp.zeros_like(w)
    top = jnp.concatenate([w, z], axis=1)
    bot = jnp.concatenate([z, w], axis=1)
    return jnp.concatenate([top, bot], axis=0)


def _prep_body(x2_ref, w1_ref, degp_ref, hp_ref):
    dis = _dis128(degp_ref)
    mm = jnp.dot(x2_ref[...], _blockdiag(w1_ref),
                 preferred_element_type=jnp.float32)
    hp_ref[0:N2, :] = mm * dis[0:N2]
    hp_ref[N2:NP2, :] = jnp.zeros((NP2 - N2, 2 * HID), jnp.float32)


def _mid_body(a_ref, hp_ref, degp_ref, b1_ref, w2_ref, hp2_ref):
    dis = _dis128(degp_ref)
    tot = a_ref[...] + hp_ref[...]
    h1 = jnp.maximum(tot * dis + b1_ref[...], 0.0)
    hp2_ref[...] = jnp.dot(h1, _blockdiag(w2_ref),
                           preferred_element_type=jnp.float32) * dis


def _head_body(a_ref, hp_ref, degp_ref, b2_ref, be_ref, bo_ref,
               wl1_ref, bl1_ref, wl2_ref, bl2_ref, out_ref):
    dis = _dis128(degp_ref)
    tot = a_ref[0:N2, :] + hp_ref[0:N2, :]
    h2 = jnp.maximum(tot * dis[0:N2] + b2_ref[...], 0.0)
    gid = lax.broadcasted_iota(jnp.int32, (N2, G), 1)
    ohe = (be_ref[...] == gid).astype(jnp.float32)
    oho = (bo_ref[...] == gid).astype(jnp.float32)
    dn = (((0,), (0,)), ((), ()))
    sums = (lax.dot_general(ohe, h2[:, 0:HID], dn,
                            preferred_element_type=jnp.float32)
            + lax.dot_general(oho, h2[:, HID:2 * HID], dn,
                              preferred_element_type=jnp.float32))
    counts = lax.dot_general(ohe + oho, jnp.ones((N2, 1), jnp.float32), dn,
                             preferred_element_type=jnp.float32)
    pooled = sums / jnp.maximum(counts, 1.0)
    h = jnp.maximum(jnp.dot(pooled, wl1_ref[...],
                            preferred_element_type=jnp.float32) + bl1_ref[...], 0.0)
    logits = jnp.dot(h, wl2_ref[...],
                     preferred_element_type=jnp.float32) + bl2_ref[...]
    out_ref[...] = 1.0 / (1.0 + jnp.exp(-logits))


def kernel(x, edge_index, batch, W1, b1, W2, b2, Wl1, bl1, Wl2, bl2):
    pad = EROWS * CH - E
    ei_pad = jnp.pad(edge_index, ((0, 0), (0, pad)), constant_values=N)
    ei_pad = ei_pad.reshape(2, EROWS, CH)
    src2d = ei_pad[0]
    dst2d = ei_pad[1]
    x2 = x.reshape(N2, 2 * IN_CH)
    b2d = batch.reshape(N2, 2)
    be = b2d[:, 0:1]
    bo = b2d[:, 1:2]
    b1p = jnp.concatenate([b1, b1]).reshape(1, 2 * HID)
    b2p = jnp.concatenate([b2, b2]).reshape(1, 2 * HID)

    degp = _make_deg_kernel()(dst2d)

    hp1 = pl.pallas_call(
        _prep_body,
        out_shape=jax.ShapeDtypeStruct((NP2, 2 * HID), jnp.float32),
    )(x2, W1, degp)

    a1 = _make_agg_kernel()(hp1.reshape(NPAD, HID), src2d, dst2d)

    hp2 = pl.pallas_call(
        _mid_body,
        out_shape=jax.ShapeDtypeStruct((NP2, 2 * HID), jnp.float32),
    )(a1.reshape(NP2, 2 * HID), hp1, degp, b1p, W2)

    a2 = _make_agg_kernel()(hp2.reshape(NPAD, HID), src2d, dst2d)

    out = pl.pallas_call(
        _head_body,
        out_shape=jax.ShapeDtypeStruct((G, 1), jnp.float32),
    )(a2.reshape(NP2, 2 * HID), hp2, degp, b2p, be, bo,
      Wl1, bl1.reshape(1, HID), Wl2, bl2.reshape(1, 1))
    return out.reshape(-1)
